# baseline (speedup 1.0000x reference)
import functools

import jax
import jax.numpy as jnp
from jax import lax
from jax.experimental import pallas as pl
from jax.experimental.pallas import tpu as pltpu
from jax.experimental.pallas import tpu_sc as plsc

NUM_EMBED = 1000000
EMBED_DIM = 64
BATCH = 16384
HIST = 50
B_TOTAL = BATCH * HIST

_info = plsc.get_sparse_core_info()
NC, NS = _info.num_cores, _info.num_subcores
NW = NC * NS
B_PER_W = B_TOTAL // NW
CHUNK = 128
NCHUNK = B_PER_W // CHUNK
NBUF = 4
DEPTH = 2


def _make_kernel():
    mesh = plsc.VectorSubcoreMesh(core_axis_name="c", subcore_axis_name="s")

    @functools.partial(
        pl.kernel,
        mesh=mesh,
        out_type=jax.ShapeDtypeStruct((HIST, 8, BATCH // CHUNK, 1024), jnp.float32),
        compiler_params=pltpu.CompilerParams(
            use_tc_tiling_on_sc=False, needs_layout_passes=False
        ),
        scratch_types=[
            pltpu.VMEM((HIST, NCHUNK // HIST, CHUNK), jnp.int32),
            pltpu.VMEM((4, 16), jnp.int32),
            [pltpu.VMEM((CHUNK, EMBED_DIM), jnp.float32) for _ in range(NBUF)],
            [pltpu.VMEM((8 * 1024,), jnp.float32) for _ in range(NBUF)],
            [pltpu.SemaphoreType.DMA for _ in range(NBUF)],
            [pltpu.SemaphoreType.DMA for _ in range(NBUF)],
        ],
    )
    def k(table_hbm, idx_hbm, out_hbm, idx_v, ovec_v, rows, blks, gsems, psems):
        wid = lax.axis_index("s") * NC + lax.axis_index("c")
        base = wid * B_PER_W
        pltpu.sync_copy(idx_hbm.at[wid], idx_v)

        dv = lax.iota(jnp.int32, 16)
        for kk in range(4):
            d = dv + (16 * kk)
            ovec_v[kk] = ((d >> 3) << 10) + ((d & 7) << 7)

        def permute(s):
            r = rows[s]
            b = blks[s]

            def pbody(c, carry):
                cv = jnp.full((16,), c, jnp.int32)
                for kk in range(4):
                    x = r[c, pl.ds(16 * kk, 16)]
                    plsc.store_scatter(b, [ovec_v[kk] + cv], x)
                return carry

            lax.fori_loop(0, CHUNK, pbody, 0)

        def fire_gather(j, s):
            h = j // (NCHUNK // HIST)
            bt = j % (NCHUNK // HIST)
            pltpu.async_copy(table_hbm.at[idx_v.at[h, bt]], rows[s], gsems[s])

        def drain_gather(s):
            pltpu.make_async_copy(
                table_hbm.at[pl.ds(0, CHUNK)], rows[s], gsems[s]
            ).wait()

        def fire_put(j, s):
            h = j // (NCHUNK // HIST)
            bt = j % (NCHUNK // HIST)
            bt0 = wid * (NCHUNK // HIST)
            for d8 in range(8):
                pltpu.async_copy(
                    blks[s].at[pl.ds(d8 * 1024, 1024)],
                    out_hbm.at[h, d8, bt0 + bt],
                    psems[s],
                )

        def drain_put(s):
            for d8 in range(8):
                pltpu.make_async_copy(
                    out_hbm.at[0, 0, 0],
                    blks[s].at[pl.ds(d8 * 1024, 1024)],
                    psems[s],
                ).wait()

        for j in range(DEPTH):
            fire_gather(j, j)

        def body(t, carry):
            for phase in range(NBUF):
                j = t * NBUF + phase
                s = phase
                sn = (phase + DEPTH) % NBUF
                jn = j + DEPTH

                @pl.when(jn < NCHUNK)
                def _():
                    @pl.when(jn >= NBUF)
                    def _():
                        drain_put(sn)

                    fire_gather(jn, sn)

                drain_gather(s)
                permute(s)
                fire_put(j, s)

            return carry

        lax.fori_loop(0, NCHUNK // NBUF, body, 0)
        for s in range(NBUF):
            drain_put(s)

    return k


_sc_gather = _make_kernel()


def kernel(inputs, vec_matrix):
    idx = (
        inputs.astype(jnp.int32)
        .reshape(NW, NCHUNK // HIST, CHUNK, HIST)
        .transpose(0, 3, 1, 2)
    )
    idx = lax.optimization_barrier(idx)
    raw = _sc_gather(vec_matrix, idx)
    out = (
        raw.reshape(HIST, 8, BATCH // CHUNK, 8, CHUNK)
        .transpose(2, 4, 0, 1, 3)
        .reshape(BATCH, HIST, EMBED_DIM)
    )
    return out


# speedup vs baseline: 1.0609x; 1.0609x over previous
import functools

import jax
import jax.numpy as jnp
from jax import lax
from jax.experimental import pallas as pl
from jax.experimental.pallas import tpu as pltpu
from jax.experimental.pallas import tpu_sc as plsc

NUM_EMBED = 1000000
EMBED_DIM = 64
BATCH = 16384
HIST = 50

_info = plsc.get_sparse_core_info()
NC, NS = _info.num_cores, _info.num_subcores
NW = NC * NS
CHUNK = 128
NBT = BATCH // (NW * CHUNK)
NCHUNK = HIST * NBT
NTILE = BATCH // CHUNK
D8 = EMBED_DIM // 8
BLK = 8 * CHUNK
NBUF = 4
DEPTH = 2
NG16 = EMBED_DIM // 16


def _make_kernel():
    mesh = plsc.VectorSubcoreMesh(core_axis_name="c", subcore_axis_name="s")

    @functools.partial(
        pl.kernel,
        mesh=mesh,
        out_type=jax.ShapeDtypeStruct((HIST, D8, NTILE, BLK), jnp.float32),
        compiler_params=pltpu.CompilerParams(
            use_tc_tiling_on_sc=False, needs_layout_passes=False
        ),
        scratch_types=[
            pltpu.VMEM((HIST, NBT, CHUNK), jnp.int32),
            pltpu.VMEM((CHUNK * NG16, 16), jnp.int32),
            [pltpu.VMEM((CHUNK, EMBED_DIM), jnp.float32) for _ in range(NBUF)],
            [pltpu.VMEM((D8 * BLK,), jnp.float32) for _ in range(NBUF)],
            [pltpu.SemaphoreType.DMA for _ in range(NBUF)],
            [pltpu.SemaphoreType.DMA for _ in range(NBUF)],
        ],
    )
    def k(table_hbm, idx_hbm, out_hbm, idx_v, pvec_v, rows, blks, gsems, psems):
        wid = lax.axis_index("s") * NC + lax.axis_index("c")
        bt0 = wid * NBT
        pltpu.sync_copy(idx_hbm.at[wid], idx_v)

        dv = lax.iota(jnp.int32, 16)
        ovecs = [
            ((d >> 3) << 10) + ((d & 7) << 7)
            for d in (dv + 16 * kk for kk in range(NG16))
        ]

        def ibody(c, carry):
            for kk in range(NG16):
                pvec_v[c * NG16 + kk] = ovecs[kk] + c
            return carry

        lax.fori_loop(0, CHUNK, ibody, 0)

        def fire_gather(g, p):
            h = g // NBT
            bt = g % NBT
            pltpu.async_copy(table_hbm.at[idx_v.at[h, bt]], rows[p], gsems[p])

        def drain_gather(p):
            pltpu.make_async_copy(
                table_hbm.at[pl.ds(0, CHUNK)], rows[p], gsems[p]
            ).wait()

        def permute(p):
            r = rows[p]
            b = blks[p]
            for c in range(CHUNK):
                for kk in range(NG16):
                    x = r[c, pl.ds(16 * kk, 16)]
                    plsc.store_scatter(b, [pvec_v[c * NG16 + kk]], x)

        def fire_put(g, p):
            h = g // NBT
            bt = g % NBT
            for d8 in range(D8):
                pltpu.async_copy(
                    blks[p].at[pl.ds(d8 * BLK, BLK)],
                    out_hbm.at[h, d8, bt0 + bt],
                    psems[p],
                )

        def drain_put(p):
            for d8 in range(D8):
                pltpu.make_async_copy(
                    out_hbm.at[0, 0, 0],
                    blks[p].at[pl.ds(d8 * BLK, BLK)],
                    psems[p],
                ).wait()

        for j in range(DEPTH):
            fire_gather(j, j)

        def body(t, carry):
            for phase in range(NBUF):
                j = t * NBUF + phase
                s = phase
                sn = (phase + DEPTH) % NBUF
                jn = j + DEPTH

                @pl.when(jn < NCHUNK)
                def _():
                    @pl.when(jn >= NBUF)
                    def _():
                        drain_put(sn)

                    fire_gather(jn, sn)

                drain_gather(s)
                permute(s)
                fire_put(j, s)

            return carry

        lax.fori_loop(0, NCHUNK // NBUF, body, 0)
        for s in range(NBUF):
            drain_put(s)

    return k


_sc_gather = _make_kernel()


def kernel(inputs, vec_matrix):
    idx = (
        inputs.astype(jnp.int32)
        .reshape(NW, NBT, CHUNK, HIST)
        .transpose(0, 3, 1, 2)
    )
    raw = _sc_gather(vec_matrix, idx)
    out = (
        raw.reshape(HIST, D8, NTILE, 8, CHUNK)
        .transpose(2, 4, 0, 1, 3)
        .reshape(BATCH, HIST, EMBED_DIM)
    )
    return out
